# baseline (speedup 1.0000x reference)
import functools

import jax
import jax.numpy as jnp
import numpy as np
from jax import lax
from jax.experimental import pallas as pl
from jax.experimental.pallas import tpu as pltpu
from jax.experimental.pallas import tpu_sc as plsc

N = 10000
E = 320000
F = 128
NREL = 16
G = 64
NCLS = 10
RN = NREL * N

NC = 2
NS = 16
NW = NC * NS
EW = E // NW

SUB = 80
KSUB = 1
CHUNK = SUB * KSUB
NCHUNK = EW // CHUNK
NROWS_E = E // SUB
RPW = NROWS_E // NW

BE = 4096
EP = 327680
BN = 2000

def _z16():
    return jnp.zeros((16,), jnp.float32)


_PERM = np.empty((F,), np.int32)
for _g in range(F // 32):
    for _i in range(16):
        _PERM[32 * _g + 2 * _i] = 32 * _g + _i
        _PERM[32 * _g + 2 * _i + 1] = 32 * _g + 16 + _i


def _edge_prep_body(attr_ref, src_ref, dst_ref, gidx_ref, sidx_ref):
    a = attr_ref[...]
    m = jnp.max(a, axis=0, keepdims=True)
    row = lax.broadcasted_iota(jnp.int32, a.shape, 0)
    et = jnp.min(jnp.where(a >= m, row, NREL), axis=0)
    gidx_ref[...] = et * N + src_ref[...]
    sidx_ref[...] = et * N + dst_ref[...]


def _edge_prep(attr_t, src, dst):
    return pl.pallas_call(
        _edge_prep_body,
        grid=(EP // BE,),
        in_specs=[
            pl.BlockSpec((NREL, BE), lambda i: (0, i)),
            pl.BlockSpec((BE,), lambda i: (i,)),
            pl.BlockSpec((BE,), lambda i: (i,)),
        ],
        out_specs=[
            pl.BlockSpec((BE,), lambda i: (i,)),
            pl.BlockSpec((BE,), lambda i: (i,)),
        ],
        out_shape=[
            jax.ShapeDtypeStruct((EP,), jnp.int32),
            jax.ShapeDtypeStruct((EP,), jnp.int32),
        ],
    )(attr_t, src, dst)


_MESH = plsc.VectorSubcoreMesh(core_axis_name="c", subcore_axis_name="s")


@functools.partial(
    pl.kernel,
    out_type=jax.ShapeDtypeStruct((NC * RN,), jnp.float32),
    mesh=_MESH,
    scratch_types=[
        pltpu.VMEM_SHARED((RN,), jnp.float32),
        pltpu.VMEM((SUB,), jnp.int32),
        pltpu.VMEM((SUB,), jnp.int32),
        pltpu.VMEM((SUB,), jnp.int32),
        pltpu.VMEM((SUB,), jnp.float32),
        pltpu.VMEM((N,), jnp.float32),
        pltpu.SemaphoreType.DMA,
        pltpu.SemaphoreType.DMA,
        pltpu.SemaphoreType.DMA,
        pltpu.SemaphoreType.DMA,
        pltpu.SemaphoreType.DMA,
        pltpu.SemaphoreType.DMA,
    ],
)
def _count_sc(sidx_hbm, cnt_hbm, cnt_sh, idx0, idx1, idx2, ones_v, z_v,
              ld0, ld1, ld2, st0, st1, st2):
    idx = (idx0, idx1, idx2)
    ld = (ld0, ld1, ld2)
    st = (st0, st1, st2)
    c = lax.axis_index("c")
    s = lax.axis_index("s")
    w = c * NS + s
    ebase = w * EW

    def zinit(i, carry):
        z_v[pl.ds(i * 16, 16)] = _z16()
        return carry

    lax.fori_loop(0, N // 16, zinit, 0)
    for i in range(SUB // 16):
        ones_v[pl.ds(i * 16, 16)] = jnp.ones((16,), jnp.float32)
    pltpu.sync_copy(z_v, cnt_sh.at[pl.ds(s * N, N)])
    plsc.subcore_barrier()

    def load_c(cc, b):
        pltpu.async_copy(sidx_hbm.at[pl.ds(ebase + cc * SUB, SUB)],
                         idx[b], ld[b])

    def wait_load(b):
        pltpu.make_async_copy(sidx_hbm.at[pl.ds(ebase, SUB)], idx[b],
                              ld[b]).wait()

    def issue_scatter(b):
        pltpu.async_copy(ones_v, cnt_sh.at[idx[b]], st[b], add=True)

    def drain_scatter(b):
        pltpu.make_async_copy(ones_v, cnt_sh.at[idx[b]], st[b]).wait()

    load_c(0, 0)
    load_c(1, 1)
    load_c(2, 2)
    wait_load(0)
    issue_scatter(0)
    wait_load(1)
    issue_scatter(1)

    def steady(k3, carry):
        for (d, b, bp) in ((2, 2, 0), (3, 0, 1), (4, 1, 2)):
            cc = 3 * k3 + d
            drain_scatter(bp)
            load_c(cc + 1, bp)
            wait_load(b)
            issue_scatter(b)
        return carry

    lax.fori_loop(0, 40, steady, 0)
    drain_scatter(0)
    load_c(123, 0)
    wait_load(2)
    issue_scatter(2)
    drain_scatter(1)
    load_c(124, 1)
    wait_load(0)
    issue_scatter(0)
    drain_scatter(2)
    wait_load(1)
    issue_scatter(1)
    drain_scatter(0)
    drain_scatter(1)
    plsc.subcore_barrier()
    pltpu.sync_copy(cnt_sh.at[pl.ds(s * N, N)], z_v)
    pltpu.sync_copy(z_v, cnt_hbm.at[pl.ds(c * RN + s * N, N)])


def _scale_body(c0_ref, c1_ref, out_ref):
    out_ref[...] = 1.0 / jnp.maximum(c0_ref[...] + c1_ref[...], 1.0)


def _scale_tc(c0, c1):
    return pl.pallas_call(
        _scale_body,
        out_shape=jax.ShapeDtypeStruct((RN,), jnp.float32),
    )(c0, c1)


def _relmm_body(h_ref, w_ref, y_ref):
    y_ref[...] = jnp.dot(h_ref[...], w_ref[0],
                         preferred_element_type=jnp.float32)


def _relmm(h, w):
    return pl.pallas_call(
        _relmm_body,
        grid=(N // BN, NREL),
        in_specs=[
            pl.BlockSpec((BN, F), lambda j, r: (j, 0)),
            pl.BlockSpec((1, F, F), lambda j, r: (r, 0, 0)),
        ],
        out_specs=pl.BlockSpec((BN, F), lambda j, r: (r * (N // BN) + j, 0)),
        out_shape=jax.ShapeDtypeStruct((RN, F), jnp.float32),
    )(h, w)


@functools.partial(
    pl.kernel,
    out_type=jax.ShapeDtypeStruct((NC, N, F), jnp.float32),
    mesh=_MESH,
    scratch_types=[
        pltpu.VMEM_SHARED((N, F), jnp.float32),
        pltpu.VMEM((EW,), jnp.int32),
        pltpu.VMEM((SUB, F), jnp.float32),
        pltpu.VMEM((SUB, F), jnp.float32),
        pltpu.VMEM((SUB, F), jnp.float32),
        pltpu.VMEM((SUB,), jnp.int32),
        pltpu.VMEM((SUB,), jnp.int32),
        pltpu.VMEM((SUB,), jnp.int32),
        pltpu.VMEM((SUB,), jnp.int32),
        pltpu.VMEM((SUB,), jnp.int32),
        pltpu.VMEM((SUB,), jnp.int32),
        pltpu.VMEM((SUB,), jnp.float32),
        pltpu.VMEM((SUB,), jnp.float32),
        pltpu.VMEM((SUB,), jnp.float32),
        pltpu.SemaphoreType.DMA,
        pltpu.SemaphoreType.DMA,
        pltpu.SemaphoreType.DMA,
        pltpu.SemaphoreType.DMA,
        pltpu.SemaphoreType.DMA,
        pltpu.SemaphoreType.DMA,
        pltpu.SemaphoreType.DMA,
        pltpu.SemaphoreType.DMA,
        pltpu.SemaphoreType.DMA,
        pltpu.SemaphoreType.DMA,
        pltpu.SemaphoreType.DMA,
        pltpu.SemaphoreType.DMA,
    ],
)
def _layer_sc(y_hbm, gidx_hbm, sidx_hbm, dst_hbm, scale_hbm, out_hbm,
              agg_sh, di_full, rows0, rows1, rows2, gi0, gi1, gi2,
              si0, si1, si2, sc0, sc1, sc2, sl0, sl1, sl2,
              sg0, sg1, sg2, ss0, ss1, ss2, st0, st1, st2):
    rows = (rows0, rows1, rows2)
    gi = (gi0, gi1, gi2)
    si = (si0, si1, si2)
    sc = (sc0, sc1, sc2)
    sl = (sl0, sl1, sl2)
    sg = (sg0, sg1, sg2)
    ss = (ss0, ss1, ss2)
    st = (st0, st1, st2)
    c = lax.axis_index("c")
    s = lax.axis_index("s")
    w = c * NS + s
    ebase = w * EW

    def zinit(i, carry):
        for k in range(F // 16):
            rows0[i, pl.ds(k * 16, 16)] = _z16()
        return carry

    lax.fori_loop(0, SUB, zinit, 0)
    for k in range(7):
        pltpu.sync_copy(rows0.at[pl.ds(0, 80)],
                        agg_sh.at[pl.ds(s * 625 + k * 80, 80)])
    pltpu.sync_copy(rows0.at[pl.ds(0, 65)],
                    agg_sh.at[pl.ds(s * 625 + 560, 65)])
    pltpu.async_copy(dst_hbm.at[pl.ds(ebase, EW)], di_full, sl0).wait()
    plsc.subcore_barrier()

    def issue_loads(cc, b):
        pltpu.async_copy(gidx_hbm.at[pl.ds(ebase + cc * SUB, SUB)],
                         gi[b], sl[b])
        pltpu.async_copy(sidx_hbm.at[pl.ds(ebase + cc * SUB, SUB)],
                         si[b], sl[b])

    def wait_loads(b):
        pltpu.make_async_copy(gidx_hbm.at[pl.ds(ebase, SUB)], gi[b],
                              sl[b]).wait()
        pltpu.make_async_copy(sidx_hbm.at[pl.ds(ebase, SUB)], si[b],
                              sl[b]).wait()

    def issue_gathers(b):
        pltpu.async_copy(y_hbm.at[gi[b]], rows[b], sg[b])
        pltpu.async_copy(scale_hbm.at[si[b]], sc[b], ss[b])

    def wait_gathers(b):
        pltpu.make_async_copy(y_hbm.at[gi[b]], rows[b], sg[b]).wait()
        pltpu.make_async_copy(scale_hbm.at[si[b]], sc[b], ss[b]).wait()

    def issue_scatter(cc, b):
        pltpu.async_copy(rows[b],
                         agg_sh.at[di_full.at[pl.ds(cc * SUB, SUB)]],
                         st[b], add=True)

    def drain_scatter(b):
        pltpu.make_async_copy(rows[b],
                              agg_sh.at[di_full.at[pl.ds(0, SUB)]],
                              st[b]).wait()

    def mult(b):
        def scale_16(q, carry):
            ev = sc[b][pl.ds(q * 16, 16)]
            for l in range(16):
                row = q * 16 + l
                for k in range(F // 16):
                    rows[b][row, pl.ds(k * 16, 16)] = (
                        rows[b][row, pl.ds(k * 16, 16)] * ev[l])
            return carry

        lax.fori_loop(0, SUB // 16, scale_16, 0)

    def proc(cc, b, bp, bn, drain, prep_ld, prep_g):
        if drain:
            drain_scatter(bp)
        if prep_ld:
            issue_loads(cc + 2, bn)
        if prep_g:
            wait_loads(bp)
            issue_gathers(bp)
        wait_gathers(b)
        mult(b)
        issue_scatter(cc, b)

    issue_loads(0, 0)
    issue_loads(1, 1)
    wait_loads(0)
    issue_gathers(0)
    proc(0, 0, 1, 2, False, True, True)
    proc(1, 1, 2, 0, False, True, True)

    def steady(k3, carry):
        for (d, b, bp, bn) in ((2, 2, 0, 1), (3, 0, 1, 2), (4, 1, 2, 0)):
            proc(3 * k3 + d, b, bp, bn, True, True, True)
        return carry

    lax.fori_loop(0, 40, steady, 0)
    proc(122, 2, 0, 1, True, True, True)
    proc(123, 0, 1, 2, True, False, True)
    proc(124, 1, 2, 0, True, False, False)
    drain_scatter(0)
    drain_scatter(1)
    plsc.subcore_barrier()

    @pl.when(s < 10)
    def _copy_out():
        r0 = s * 1000
        for k in range(13):
            o = k * 80
            nr = 80 if k < 12 else 40
            pltpu.sync_copy(agg_sh.at[pl.ds(r0 + o, nr)],
                            rows0.at[pl.ds(0, nr)])
            pltpu.sync_copy(rows0.at[pl.ds(0, nr)],
                            out_hbm.at[c, pl.ds(r0 + o, nr)])


def _epi_body(x_ref, p0_ref, p1_ref, root_ref, b_ref, h_ref):
    acc = jnp.dot(x_ref[...], root_ref[...],
                  preferred_element_type=jnp.float32)
    h_ref[...] = jnp.maximum(acc + p0_ref[...] + p1_ref[...] + b_ref[...],
                             0.0)


def _epi_tc(x, p0, p1, root, b2d):
    return pl.pallas_call(
        _epi_body,
        grid=(N // BN,),
        in_specs=[
            pl.BlockSpec((BN, F), lambda j: (j, 0)),
            pl.BlockSpec((BN, F), lambda j: (j, 0)),
            pl.BlockSpec((BN, F), lambda j: (j, 0)),
            pl.BlockSpec((F, F), lambda j: (0, 0)),
            pl.BlockSpec((1, F), lambda j: (0, 0)),
        ],
        out_specs=pl.BlockSpec((BN, F), lambda j: (j, 0)),
        out_shape=jax.ShapeDtypeStruct((N, F), jnp.float32),
    )(x, p0, p1, root, b2d)


def _head_body(h_ref, batch_ref, w1_ref, b1_ref, w2_ref, b2_ref, w3_ref,
               b3_ref, out_ref):
    h = h_ref[...]
    bt = batch_ref[...]
    mt = (lax.broadcasted_iota(jnp.int32, (G, N), 0) == bt)
    m = mt.astype(jnp.float32)
    cnt = jnp.sum(m, axis=1)
    gs = jnp.dot(m, h, preferred_element_type=jnp.float32)
    g = gs / jnp.maximum(cnt, 1.0)[:, None]
    g = jnp.maximum(jnp.dot(g, w1_ref[...],
                            preferred_element_type=jnp.float32)
                    + b1_ref[...], 0.0)
    g = jnp.maximum(jnp.dot(g, w2_ref[...],
                            preferred_element_type=jnp.float32)
                    + b2_ref[...], 0.0)
    logits = jnp.dot(g, w3_ref[...],
                     preferred_element_type=jnp.float32) + b3_ref[...]
    mx = jnp.max(logits, axis=1, keepdims=True)
    lse = jnp.log(jnp.sum(jnp.exp(logits - mx), axis=1, keepdims=True)) + mx
    out_ref[...] = logits - lse


def _head_tc(h, batch2d, w1, b1, w2, b2, w3p, b3p):
    return pl.pallas_call(
        _head_body,
        out_shape=jax.ShapeDtypeStruct((G, F), jnp.float32),
    )(h, batch2d, w1, b1, w2, b2, w3p, b3p)


def kernel(x, edge_index, edge_attr, batch, W1, root1, b1, W2, root2, b2,
           lin1_w, lin1_b, lin2_w, lin2_b, lin3_w, lin3_b):
    src = edge_index[0]
    dst = edge_index[1]
    attr_t = jnp.pad(edge_attr.T, ((0, 0), (0, EP - E)))
    gidx, sidx = _edge_prep(attr_t, jnp.pad(src, (0, EP - E)),
                            jnp.pad(dst, (0, EP - E)))

    cnt = _count_sc(sidx)
    scale_tab = _scale_tc(cnt[:RN], cnt[RN:])

    y1 = _relmm(x, W1)
    p1 = _layer_sc(y1, gidx, sidx, dst, scale_tab)
    h = _epi_tc(x, p1[0], p1[1], root1, b1.reshape(1, F))

    y2 = _relmm(h, W2)
    p2 = _layer_sc(y2, gidx, sidx, dst, scale_tab)
    h2 = _epi_tc(h, p2[0], p2[1], root2, b2.reshape(1, F))

    w3p = jnp.pad(lin3_w, ((0, 0), (0, F - NCLS)))
    b3p = jnp.concatenate(
        [lin3_b, jnp.full((F - NCLS,), -1e30, jnp.float32)]).reshape(1, F)
    out = _head_tc(h2, batch.reshape(1, N), lin1_w, lin1_b.reshape(1, F),
                   lin2_w, lin2_b.reshape(1, F // 2), w3p, b3p)
    return out[:, :NCLS]


# speedup vs baseline: 1.0017x; 1.0017x over previous
import functools

import jax
import jax.numpy as jnp
from jax import lax
from jax.experimental import pallas as pl
from jax.experimental.pallas import tpu as pltpu
from jax.experimental.pallas import tpu_sc as plsc

N = 10000
E = 320000
F = 128
NREL = 16
G = 64
NCLS = 10
RN = NREL * N

NC = 2
NS = 16
NW = NC * NS
EW = E // NW

SUB = 80
KSUB = 1
CHUNK = SUB * KSUB
NCHUNK = EW // CHUNK
NROWS_E = E // SUB
RPW = NROWS_E // NW

BE = 4096
EP = 327680
BN = 2000

def _z16():
    return jnp.zeros((16,), jnp.float32)


def _edge_prep_body(attr_ref, src_ref, dst_ref, gidx_ref, sidx_ref):
    a = attr_ref[...]
    m = jnp.max(a, axis=0, keepdims=True)
    row = lax.broadcasted_iota(jnp.int32, a.shape, 0)
    et = jnp.min(jnp.where(a >= m, row, NREL), axis=0)
    gidx_ref[...] = et * N + src_ref[...]
    sidx_ref[...] = et * N + dst_ref[...]


def _edge_prep(attr_t, src, dst):
    return pl.pallas_call(
        _edge_prep_body,
        grid=(EP // BE,),
        in_specs=[
            pl.BlockSpec((NREL, BE), lambda i: (0, i)),
            pl.BlockSpec((BE,), lambda i: (i,)),
            pl.BlockSpec((BE,), lambda i: (i,)),
        ],
        out_specs=[
            pl.BlockSpec((BE,), lambda i: (i,)),
            pl.BlockSpec((BE,), lambda i: (i,)),
        ],
        out_shape=[
            jax.ShapeDtypeStruct((EP,), jnp.int32),
            jax.ShapeDtypeStruct((EP,), jnp.int32),
        ],
    )(attr_t, src, dst)


_MESH = plsc.VectorSubcoreMesh(core_axis_name="c", subcore_axis_name="s")


@functools.partial(
    pl.kernel,
    out_type=jax.ShapeDtypeStruct((NC * RN,), jnp.float32),
    mesh=_MESH,
    scratch_types=[
        pltpu.VMEM_SHARED((RN,), jnp.float32),
        pltpu.VMEM((SUB,), jnp.int32),
        pltpu.VMEM((SUB,), jnp.int32),
        pltpu.VMEM((SUB,), jnp.int32),
        pltpu.VMEM((SUB,), jnp.float32),
        pltpu.VMEM((N,), jnp.float32),
        pltpu.SemaphoreType.DMA,
        pltpu.SemaphoreType.DMA,
        pltpu.SemaphoreType.DMA,
        pltpu.SemaphoreType.DMA,
        pltpu.SemaphoreType.DMA,
        pltpu.SemaphoreType.DMA,
    ],
)
def _count_sc(sidx_hbm, cnt_hbm, cnt_sh, idx0, idx1, idx2, ones_v, z_v,
              ld0, ld1, ld2, st0, st1, st2):
    idx = (idx0, idx1, idx2)
    ld = (ld0, ld1, ld2)
    st = (st0, st1, st2)
    c = lax.axis_index("c")
    s = lax.axis_index("s")
    w = c * NS + s
    ebase = w * EW

    def zinit(i, carry):
        z_v[pl.ds(i * 16, 16)] = _z16()
        return carry

    lax.fori_loop(0, N // 16, zinit, 0)
    for i in range(SUB // 16):
        ones_v[pl.ds(i * 16, 16)] = jnp.ones((16,), jnp.float32)
    pltpu.sync_copy(z_v, cnt_sh.at[pl.ds(s * N, N)])
    plsc.subcore_barrier()

    def load_c(cc, b):
        pltpu.async_copy(sidx_hbm.at[pl.ds(ebase + cc * SUB, SUB)],
                         idx[b], ld[b])

    def wait_load(b):
        pltpu.make_async_copy(sidx_hbm.at[pl.ds(ebase, SUB)], idx[b],
                              ld[b]).wait()

    def issue_scatter(b):
        pltpu.async_copy(ones_v, cnt_sh.at[idx[b]], st[b], add=True)

    def drain_scatter(b):
        pltpu.make_async_copy(ones_v, cnt_sh.at[idx[b]], st[b]).wait()

    load_c(0, 0)
    load_c(1, 1)
    load_c(2, 2)
    wait_load(0)
    issue_scatter(0)
    wait_load(1)
    issue_scatter(1)

    def steady(k3, carry):
        for (d, b, bp) in ((2, 2, 0), (3, 0, 1), (4, 1, 2)):
            cc = 3 * k3 + d
            drain_scatter(bp)
            load_c(cc + 1, bp)
            wait_load(b)
            issue_scatter(b)
        return carry

    lax.fori_loop(0, 40, steady, 0)
    drain_scatter(0)
    load_c(123, 0)
    wait_load(2)
    issue_scatter(2)
    drain_scatter(1)
    load_c(124, 1)
    wait_load(0)
    issue_scatter(0)
    drain_scatter(2)
    wait_load(1)
    issue_scatter(1)
    drain_scatter(0)
    drain_scatter(1)
    plsc.subcore_barrier()
    pltpu.sync_copy(cnt_sh.at[pl.ds(s * N, N)], z_v)
    pltpu.sync_copy(z_v, cnt_hbm.at[pl.ds(c * RN + s * N, N)])


def _scale_body(c0_ref, c1_ref, out_ref):
    out_ref[...] = 1.0 / jnp.maximum(c0_ref[...] + c1_ref[...], 1.0)


def _scale_tc(c0, c1):
    return pl.pallas_call(
        _scale_body,
        out_shape=jax.ShapeDtypeStruct((RN,), jnp.float32),
    )(c0, c1)


def _relmm_body(h_ref, w_ref, y_ref):
    y_ref[...] = jnp.dot(h_ref[...], w_ref[0],
                         preferred_element_type=jnp.float32)


def _relmm(h, w):
    return pl.pallas_call(
        _relmm_body,
        grid=(N // BN, NREL),
        in_specs=[
            pl.BlockSpec((BN, F), lambda j, r: (j, 0)),
            pl.BlockSpec((1, F, F), lambda j, r: (r, 0, 0)),
        ],
        out_specs=pl.BlockSpec((BN, F), lambda j, r: (r * (N // BN) + j, 0)),
        out_shape=jax.ShapeDtypeStruct((RN, F), jnp.float32),
    )(h, w)


@functools.partial(
    pl.kernel,
    out_type=jax.ShapeDtypeStruct((NC, N, F), jnp.float32),
    mesh=_MESH,
    scratch_types=[
        pltpu.VMEM_SHARED((N, F), jnp.float32),
        pltpu.VMEM((EW,), jnp.int32),
        pltpu.VMEM((SUB, F), jnp.float32),
        pltpu.VMEM((SUB, F), jnp.float32),
        pltpu.VMEM((SUB, F), jnp.float32),
        pltpu.VMEM((SUB,), jnp.int32),
        pltpu.VMEM((SUB,), jnp.int32),
        pltpu.VMEM((SUB,), jnp.int32),
        pltpu.VMEM((SUB,), jnp.int32),
        pltpu.VMEM((SUB,), jnp.int32),
        pltpu.VMEM((SUB,), jnp.int32),
        pltpu.VMEM((SUB,), jnp.float32),
        pltpu.VMEM((SUB,), jnp.float32),
        pltpu.VMEM((SUB,), jnp.float32),
        pltpu.SemaphoreType.DMA,
        pltpu.SemaphoreType.DMA,
        pltpu.SemaphoreType.DMA,
        pltpu.SemaphoreType.DMA,
        pltpu.SemaphoreType.DMA,
        pltpu.SemaphoreType.DMA,
        pltpu.SemaphoreType.DMA,
        pltpu.SemaphoreType.DMA,
        pltpu.SemaphoreType.DMA,
        pltpu.SemaphoreType.DMA,
        pltpu.SemaphoreType.DMA,
        pltpu.SemaphoreType.DMA,
    ],
)
def _layer_sc(y_hbm, gidx_hbm, sidx_hbm, dst_hbm, scale_hbm, out_hbm,
              agg_sh, di_full, rows0, rows1, rows2, gi0, gi1, gi2,
              si0, si1, si2, sc0, sc1, sc2, sl0, sl1, sl2,
              sg0, sg1, sg2, ss0, ss1, ss2, st0, st1, st2):
    rows = (rows0, rows1, rows2)
    gi = (gi0, gi1, gi2)
    si = (si0, si1, si2)
    sc = (sc0, sc1, sc2)
    sl = (sl0, sl1, sl2)
    sg = (sg0, sg1, sg2)
    ss = (ss0, ss1, ss2)
    st = (st0, st1, st2)
    c = lax.axis_index("c")
    s = lax.axis_index("s")
    w = c * NS + s
    ebase = w * EW

    def zinit(i, carry):
        for k in range(F // 16):
            rows0[i, pl.ds(k * 16, 16)] = _z16()
        return carry

    lax.fori_loop(0, SUB, zinit, 0)
    for k in range(7):
        pltpu.sync_copy(rows0.at[pl.ds(0, 80)],
                        agg_sh.at[pl.ds(s * 625 + k * 80, 80)])
    pltpu.sync_copy(rows0.at[pl.ds(0, 65)],
                    agg_sh.at[pl.ds(s * 625 + 560, 65)])
    pltpu.async_copy(dst_hbm.at[pl.ds(ebase, EW)], di_full, sl0).wait()
    plsc.subcore_barrier()

    def issue_loads(cc, b):
        pltpu.async_copy(gidx_hbm.at[pl.ds(ebase + cc * SUB, SUB)],
                         gi[b], sl[b])
        pltpu.async_copy(sidx_hbm.at[pl.ds(ebase + cc * SUB, SUB)],
                         si[b], sl[b])

    def wait_loads(b):
        pltpu.make_async_copy(gidx_hbm.at[pl.ds(ebase, SUB)], gi[b],
                              sl[b]).wait()
        pltpu.make_async_copy(sidx_hbm.at[pl.ds(ebase, SUB)], si[b],
                              sl[b]).wait()

    def issue_gathers(b):
        pltpu.async_copy(y_hbm.at[gi[b]], rows[b], sg[b])
        pltpu.async_copy(scale_hbm.at[si[b]], sc[b], ss[b])

    def wait_gathers(b):
        pltpu.make_async_copy(y_hbm.at[gi[b]], rows[b], sg[b]).wait()
        pltpu.make_async_copy(scale_hbm.at[si[b]], sc[b], ss[b]).wait()

    def issue_scatter(cc, b):
        pltpu.async_copy(rows[b],
                         agg_sh.at[di_full.at[pl.ds(cc * SUB, SUB)]],
                         st[b], add=True)

    def drain_scatter(b):
        pltpu.make_async_copy(rows[b],
                              agg_sh.at[di_full.at[pl.ds(0, SUB)]],
                              st[b]).wait()

    def mult(b):
        def scale_16(q, carry):
            ev = sc[b][pl.ds(q * 16, 16)]
            for l in range(16):
                row = q * 16 + l
                for k in range(F // 16):
                    rows[b][row, pl.ds(k * 16, 16)] = (
                        rows[b][row, pl.ds(k * 16, 16)] * ev[l])
            return carry

        lax.fori_loop(0, SUB // 16, scale_16, 0)

    def proc(cc, b, bp, bn, drain, prep_ld, prep_g):
        if drain:
            drain_scatter(bp)
        if prep_ld:
            issue_loads(cc + 2, bn)
        if prep_g:
            wait_loads(bp)
            issue_gathers(bp)
        wait_gathers(b)
        mult(b)
        issue_scatter(cc, b)

    issue_loads(0, 0)
    issue_loads(1, 1)
    wait_loads(0)
    issue_gathers(0)
    proc(0, 0, 1, 2, False, True, True)
    proc(1, 1, 2, 0, False, True, True)

    def steady(k3, carry):
        for (d, b, bp, bn) in ((2, 2, 0, 1), (3, 0, 1, 2), (4, 1, 2, 0)):
            proc(3 * k3 + d, b, bp, bn, True, True, True)
        return carry

    lax.fori_loop(0, 40, steady, 0)
    proc(122, 2, 0, 1, True, True, True)
    proc(123, 0, 1, 2, True, False, True)
    proc(124, 1, 2, 0, True, False, False)
    drain_scatter(0)
    drain_scatter(1)
    plsc.subcore_barrier()

    @pl.when(s < 10)
    def _copy_out():
        r0 = s * 1000
        for k in range(13):
            o = k * 80
            nr = 80 if k < 12 else 40
            pltpu.sync_copy(agg_sh.at[pl.ds(r0 + o, nr)],
                            rows0.at[pl.ds(0, nr)])
            pltpu.sync_copy(rows0.at[pl.ds(0, nr)],
                            out_hbm.at[c, pl.ds(r0 + o, nr)])


def _epi_body(x_ref, p0_ref, p1_ref, root_ref, b_ref, h_ref):
    acc = jnp.dot(x_ref[...], root_ref[...],
                  preferred_element_type=jnp.float32)
    h_ref[...] = jnp.maximum(acc + p0_ref[...] + p1_ref[...] + b_ref[...],
                             0.0)


def _epi_tc(x, p0, p1, root, b2d):
    return pl.pallas_call(
        _epi_body,
        grid=(N // BN,),
        in_specs=[
            pl.BlockSpec((BN, F), lambda j: (j, 0)),
            pl.BlockSpec((BN, F), lambda j: (j, 0)),
            pl.BlockSpec((BN, F), lambda j: (j, 0)),
            pl.BlockSpec((F, F), lambda j: (0, 0)),
            pl.BlockSpec((1, F), lambda j: (0, 0)),
        ],
        out_specs=pl.BlockSpec((BN, F), lambda j: (j, 0)),
        out_shape=jax.ShapeDtypeStruct((N, F), jnp.float32),
    )(x, p0, p1, root, b2d)


def _head_body(h_ref, batch_ref, w1_ref, b1_ref, w2_ref, b2_ref, w3_ref,
               b3_ref, out_ref):
    h = h_ref[...]
    bt = batch_ref[...]
    mt = (lax.broadcasted_iota(jnp.int32, (G, N), 0) == bt)
    m = mt.astype(jnp.float32)
    cnt = jnp.sum(m, axis=1)
    gs = jnp.dot(m, h, preferred_element_type=jnp.float32)
    g = gs / jnp.maximum(cnt, 1.0)[:, None]
    g = jnp.maximum(jnp.dot(g, w1_ref[...],
                            preferred_element_type=jnp.float32)
                    + b1_ref[...], 0.0)
    g = jnp.maximum(jnp.dot(g, w2_ref[...],
                            preferred_element_type=jnp.float32)
                    + b2_ref[...], 0.0)
    logits = jnp.dot(g, w3_ref[...],
                     preferred_element_type=jnp.float32) + b3_ref[...]
    mx = jnp.max(logits, axis=1, keepdims=True)
    lse = jnp.log(jnp.sum(jnp.exp(logits - mx), axis=1, keepdims=True)) + mx
    out_ref[...] = logits - lse


def _head_tc(h, batch2d, w1, b1, w2, b2, w3p, b3p):
    return pl.pallas_call(
        _head_body,
        out_shape=jax.ShapeDtypeStruct((G, F), jnp.float32),
    )(h, batch2d, w1, b1, w2, b2, w3p, b3p)


def kernel(x, edge_index, edge_attr, batch, W1, root1, b1, W2, root2, b2,
           lin1_w, lin1_b, lin2_w, lin2_b, lin3_w, lin3_b):
    src = edge_index[0]
    dst = edge_index[1]
    attr_t = jnp.pad(edge_attr.T, ((0, 0), (0, EP - E)))
    gidx, sidx = _edge_prep(attr_t, jnp.pad(src, (0, EP - E)),
                            jnp.pad(dst, (0, EP - E)))

    cnt = _count_sc(sidx)
    scale_tab = _scale_tc(cnt[:RN], cnt[RN:])

    y1 = _relmm(x, W1)
    p1 = _layer_sc(y1, gidx, sidx, dst, scale_tab)
    h = _epi_tc(x, p1[0], p1[1], root1, b1.reshape(1, F))

    y2 = _relmm(h, W2)
    p2 = _layer_sc(y2, gidx, sidx, dst, scale_tab)
    h2 = _epi_tc(h, p2[0], p2[1], root2, b2.reshape(1, F))

    w3p = jnp.pad(lin3_w, ((0, 0), (0, F - NCLS)))
    b3p = jnp.concatenate(
        [lin3_b, jnp.full((F - NCLS,), -1e30, jnp.float32)]).reshape(1, F)
    out = _head_tc(h2, batch.reshape(1, N), lin1_w, lin1_b.reshape(1, F),
                   lin2_w, lin2_b.reshape(1, F // 2), w3p, b3p)
    return out[:, :NCLS]
